# Initial kernel scaffold; baseline (speedup 1.0000x reference)
import jax
import jax.numpy as jnp
from jax.experimental import pallas as pl


def kernel(input_ids, wte):
    raise NotImplementedError("write your pallas kernel here")


# speedup vs baseline: 1.8294x; 1.8294x over previous
import functools

import jax
import jax.numpy as jnp
from jax import lax
from jax.experimental import pallas as pl
from jax.experimental.pallas import tpu as pltpu
from jax.experimental.pallas import tpu_sc as plsc

VOCAB = 1000000
N_EMBD = 64
BATCH = 16384
SEQ = 50

B = BATCH * SEQ

_info = plsc.get_sparse_core_info()
NC = _info.num_cores
NS = _info.num_subcores
NW = NC * NS

B_PER_W = B // NW
CHUNK = 800
N_CHUNKS = B_PER_W // CHUNK


def _body(ids_hbm, table_hbm, out_hbm, idx_v, rows_v, sem):
    wid = lax.axis_index("s") * NC + lax.axis_index("c")
    base = wid * B_PER_W

    def chunk(j, carry):
        off = base + j * CHUNK
        pltpu.sync_copy(ids_hbm.at[pl.ds(off, CHUNK)], idx_v)
        pltpu.async_copy(table_hbm.at[idx_v], rows_v, sem).wait()
        pltpu.sync_copy(rows_v, out_hbm.at[pl.ds(off, CHUNK)])
        return carry

    lax.fori_loop(0, N_CHUNKS, chunk, 0)


@jax.jit
def kernel(input_ids, wte):
    ids_flat = input_ids.reshape(-1).astype(jnp.int32)
    mesh = plsc.VectorSubcoreMesh(core_axis_name="c", subcore_axis_name="s")
    out = pl.kernel(
        _body,
        out_type=jax.ShapeDtypeStruct((B, N_EMBD), jnp.float32),
        mesh=mesh,
        scratch_types=[
            pltpu.VMEM((CHUNK,), jnp.int32),
            pltpu.VMEM((CHUNK, N_EMBD), jnp.float32),
            pltpu.SemaphoreType.DMA,
        ],
        compiler_params=pltpu.CompilerParams(use_tc_tiling_on_sc=False),
    )(ids_flat, wte)
    return out.reshape(BATCH, SEQ, N_EMBD)


# speedup vs baseline: 1.8850x; 1.0304x over previous
import jax
import jax.numpy as jnp
from jax import lax
from jax.experimental import pallas as pl
from jax.experimental.pallas import tpu as pltpu
from jax.experimental.pallas import tpu_sc as plsc

VOCAB = 1000000
N_EMBD = 64
BATCH = 16384
SEQ = 50

B = BATCH * SEQ

_info = plsc.get_sparse_core_info()
NC = _info.num_cores
NS = _info.num_subcores
NW = NC * NS

B_PER_W = B // NW
CHUNK = 400
NBUF = 4
N_CHUNKS = B_PER_W // CHUNK


def _body(ids_hbm, table_hbm, out_hbm, idx_v, rows0, rows1, rows2, rows3,
          sg0, sg1, sg2, sg3, so0, so1, so2, so3):
    rows = (rows0, rows1, rows2, rows3)
    sg = (sg0, sg1, sg2, sg3)
    so = (so0, so1, so2, so3)

    wid = lax.axis_index("s") * NC + lax.axis_index("c")
    base = wid * B_PER_W
    pltpu.sync_copy(ids_hbm.at[pl.ds(base, B_PER_W)], idx_v)

    def gather_start(chunk, b):
        pltpu.async_copy(
            table_hbm.at[idx_v.at[pl.ds(chunk * CHUNK, CHUNK)]], rows[b], sg[b])

    for b in range(NBUF - 1):
        gather_start(b, b)

    def step(outer, carry):
        for bb in range(NBUF):
            j = outer + bb

            pltpu.make_async_copy(
                table_hbm.at[idx_v.at[pl.ds(0, CHUNK)]], rows[bb], sg[bb]).wait()
            pltpu.async_copy(rows[bb], out_hbm.at[pl.ds(base + j * CHUNK, CHUNK)],
                             so[bb])

            nb = (bb + NBUF - 1) % NBUF
            nxt = j + NBUF - 1

            @pl.when(j >= 1)
            def _():
                pltpu.make_async_copy(
                    rows[nb], out_hbm.at[pl.ds(0, CHUNK)], so[nb]).wait()

            @pl.when(nxt < N_CHUNKS)
            def _():
                pltpu.async_copy(
                    table_hbm.at[idx_v.at[pl.ds(nxt * CHUNK, CHUNK)]],
                    rows[nb], sg[nb])
        return carry

    lax.fori_loop(0, N_CHUNKS // NBUF, lambda i, c: step(i * NBUF, c), 0,
                  unroll=False)

    lb = (N_CHUNKS - 1) % NBUF
    pltpu.make_async_copy(rows[lb], out_hbm.at[pl.ds(0, CHUNK)], so[lb]).wait()


@jax.jit
def kernel(input_ids, wte):
    ids_flat = input_ids.reshape(-1).astype(jnp.int32)
    mesh = plsc.VectorSubcoreMesh(core_axis_name="c", subcore_axis_name="s")
    out = pl.kernel(
        _body,
        out_type=jax.ShapeDtypeStruct((B, N_EMBD), jnp.float32),
        mesh=mesh,
        scratch_types=(
            [pltpu.VMEM((B_PER_W,), jnp.int32)]
            + [pltpu.VMEM((CHUNK, N_EMBD), jnp.float32) for _ in range(NBUF)]
            + [pltpu.SemaphoreType.DMA for _ in range(2 * NBUF)]
        ),
        compiler_params=pltpu.CompilerParams(use_tc_tiling_on_sc=False),
    )(ids_flat, wte)
    return out.reshape(BATCH, SEQ, N_EMBD)
